# Initial kernel scaffold; baseline (speedup 1.0000x reference)
import jax
import jax.numpy as jnp
from jax.experimental import pallas as pl


def kernel(node_feature, edge_index, vectors, params):
    raise NotImplementedError("write your pallas kernel here")


# speedup vs baseline: 2.1825x; 2.1825x over previous
import functools

import jax
import jax.numpy as jnp
from jax import lax
from jax.experimental import pallas as pl
from jax.experimental.pallas import tpu as pltpu
from jax.experimental.pallas import tpu_sc as plsc

N_NODES = 10000
N_EDGES = 160000
D_FEAT = 300
DH = 100
DP = 128
NC, NS = 2, 16
NW = NC * NS
EPW = N_EDGES // NW
CHUNK = 128
NFULL = EPW // CHUNK
TAIL = EPW - NFULL * CHUNK
OWN = 6000
TROWS = 6016
TSTRIPE = TROWS // NS
TRASH = TROWS - 1
EPT = N_EDGES // NS
NF2 = EPT // CHUNK
TAIL2 = EPT - NF2 * CHUNK
BE = 2000
BN = 2000
EPS = 1e-5

f32 = jnp.float32


def _swish(x):
    return x * lax.logistic(x)


def _pad2(w, r, c):
    return jnp.zeros((r, c), f32).at[: w.shape[0], : w.shape[1]].set(w)


def _pad1(b, n):
    return jnp.zeros((1, n), f32).at[0, : b.shape[0]].set(b)


def _node_pre_body(x_ref, wa_ref, wb_ref, b_ref, a_ref, bb_ref):
    xb = x_ref[...]
    a_ref[...] = jnp.dot(xb, wa_ref[...], preferred_element_type=f32) + b_ref[...]
    bb_ref[...] = jnp.dot(xb[:, :DH], wb_ref[...], preferred_element_type=f32)


def _node_pre(x, wa, wb, b):
    g = N_NODES // BN
    return pl.pallas_call(
        _node_pre_body,
        grid=(g,),
        in_specs=[
            pl.BlockSpec((BN, D_FEAT), lambda i: (i, 0)),
            pl.BlockSpec((D_FEAT, DP), lambda i: (0, 0)),
            pl.BlockSpec((DH, DP), lambda i: (0, 0)),
            pl.BlockSpec((1, DP), lambda i: (0, 0)),
        ],
        out_specs=[
            pl.BlockSpec((BN, DP), lambda i: (i, 0)),
            pl.BlockSpec((BN, DP), lambda i: (i, 0)),
        ],
        out_shape=[
            jax.ShapeDtypeStruct((N_NODES, DP), f32),
            jax.ShapeDtypeStruct((N_NODES, DP), f32),
        ],
    )(x, wa, wb, b)


def _s1_body(e_ref, sum_ref, sq_ref):
    i = pl.program_id(0)
    s = _swish(e_ref[...])
    ps = jnp.sum(s, axis=0, keepdims=True)
    pq = jnp.sum(s * s, axis=0, keepdims=True)

    @pl.when(i == 0)
    def _():
        sum_ref[...] = ps
        sq_ref[...] = pq

    @pl.when(i != 0)
    def _():
        sum_ref[...] = sum_ref[...] + ps
        sq_ref[...] = sq_ref[...] + pq


def _s1_stats(e1):
    g = N_EDGES // BE
    return pl.pallas_call(
        _s1_body,
        grid=(g,),
        in_specs=[pl.BlockSpec((BE, DP), lambda i: (i, 0))],
        out_specs=[
            pl.BlockSpec((1, DP), lambda i: (0, 0)),
            pl.BlockSpec((1, DP), lambda i: (0, 0)),
        ],
        out_shape=[
            jax.ShapeDtypeStruct((1, DP), f32),
            jax.ShapeDtypeStruct((1, DP), f32),
        ],
    )(e1)


def _stage_body(in_ref, sum_ref, sq_ref, g_ref, be_ref, w_ref, b_ref,
                out_ref, osum_ref, osq_ref, *, pre_swish, track_stats, ones_col):
    i = pl.program_id(0)
    h = in_ref[...]
    if pre_swish:
        h = _swish(h)
    m = sum_ref[...] * (1.0 / N_EDGES)
    var = sq_ref[...] * (1.0 / N_EDGES) - m * m
    h = (h - m) * lax.rsqrt(var + EPS) * g_ref[...] + be_ref[...]
    z = jnp.dot(h, w_ref[...], preferred_element_type=f32) + b_ref[...]
    s = _swish(z)
    if ones_col:
        col = lax.broadcasted_iota(jnp.int32, s.shape, 1)
        s = jnp.where(col == DP - 1, 1.0, s)
    out_ref[...] = s
    if track_stats:
        ps = jnp.sum(s, axis=0, keepdims=True)
        pq = jnp.sum(s * s, axis=0, keepdims=True)

        @pl.when(i == 0)
        def _():
            osum_ref[...] = ps
            osq_ref[...] = pq

        @pl.when(i != 0)
        def _():
            osum_ref[...] = osum_ref[...] + ps
            osq_ref[...] = osq_ref[...] + pq


def _stage(x, stats, gamma, beta, w, b, *, pre_swish, track_stats, ones_col=False):
    g = N_EDGES // BE
    one = pl.BlockSpec((1, DP), lambda i: (0, 0))
    body = functools.partial(_stage_body, pre_swish=pre_swish,
                             track_stats=track_stats, ones_col=ones_col)
    out_specs = [pl.BlockSpec((BE, DP), lambda i: (i, 0)), one, one]
    out_shape = [
        jax.ShapeDtypeStruct((N_EDGES, DP), f32),
        jax.ShapeDtypeStruct((1, DP), f32),
        jax.ShapeDtypeStruct((1, DP), f32),
    ]
    if not track_stats:
        out_specs, out_shape = out_specs[:1], out_shape[:1]
    res = pl.pallas_call(
        body if track_stats else (lambda a, s1, s2, gg, bb, ww, bc, o:
                                  _stage_body(a, s1, s2, gg, bb, ww, bc, o, None, None,
                                              pre_swish=pre_swish, track_stats=False,
                                              ones_col=ones_col)),
        grid=(g,),
        in_specs=[pl.BlockSpec((BE, DP), lambda i: (i, 0)), one, one, one, one,
                  pl.BlockSpec((DP, DP), lambda i: (0, 0)), one],
        out_specs=out_specs,
        out_shape=out_shape,
    )(x, stats[0], stats[1], gamma, beta, w, b)
    return res


def _update_body(x_ref, v_ref, a_ref, wv_ref, wx_ref, wa_ref, b1_ref,
                 w2_ref, b2_ref, w3_ref, b3_ref, w4_ref, b4_ref, out_ref):
    xb = x_ref[...]
    acc = a_ref[0]
    cnt = jnp.maximum(acc[:, DP - 1 : DP], 1.0)
    agg = acc / cnt
    h = (jnp.dot(v_ref[...], wv_ref[...], preferred_element_type=f32)
         + jnp.dot(xb, wx_ref[...], preferred_element_type=f32)
         + jnp.dot(agg, wa_ref[...], preferred_element_type=f32)
         + b1_ref[...])
    h = _swish(h)
    h = _swish(jnp.dot(h, w2_ref[...], preferred_element_type=f32) + b2_ref[...])
    h = _swish(jnp.dot(h, w3_ref[...], preferred_element_type=f32) + b3_ref[...])
    upd = _swish(jnp.dot(h, w4_ref[...], preferred_element_type=f32) + b4_ref[...])
    out_ref[...] = xb + upd


def _update(x, v, aggs, wv, wx, wa, b1, w2, b2, w3, b3, w4, b4):
    g = N_NODES // BN
    oneh = pl.BlockSpec((1, DH), lambda i: (0, 0))
    return pl.pallas_call(
        _update_body,
        grid=(g,),
        in_specs=[
            pl.BlockSpec((BN, D_FEAT), lambda i: (i, 0)),
            pl.BlockSpec((BN, DH), lambda i: (i, 0)),
            pl.BlockSpec((1, BN, DP), lambda i: (i // 3, i - 3 * (i // 3), 0)),
            pl.BlockSpec((DH, DH), lambda i: (0, 0)),
            pl.BlockSpec((D_FEAT, DH), lambda i: (0, 0)),
            pl.BlockSpec((DP, DH), lambda i: (0, 0)),
            oneh,
            pl.BlockSpec((DH, DH), lambda i: (0, 0)), oneh,
            pl.BlockSpec((DH, DH), lambda i: (0, 0)), oneh,
            pl.BlockSpec((DH, D_FEAT), lambda i: (0, 0)),
            pl.BlockSpec((1, D_FEAT), lambda i: (0, 0)),
        ],
        out_specs=[pl.BlockSpec((BN, D_FEAT), lambda i: (i, 0))],
        out_shape=[jax.ShapeDtypeStruct((N_NODES, D_FEAT), f32)],
    )(x, v, aggs, wv, wx, wa, b1, w2, b2, w3, b3, w4, b4)[0]


def _pre_pool_body(x_ref, w1_ref, b1_ref, w2_ref, b2_ref, w3_ref, b3_ref,
                   w4_ref, b4_ref, sum_ref):
    i = pl.program_id(0)
    h = _swish(jnp.dot(x_ref[...], w1_ref[...], preferred_element_type=f32) + b1_ref[...])
    h = _swish(jnp.dot(h, w2_ref[...], preferred_element_type=f32) + b2_ref[...])
    h = _swish(jnp.dot(h, w3_ref[...], preferred_element_type=f32) + b3_ref[...])
    h = jnp.dot(h, w4_ref[...], preferred_element_type=f32) + b4_ref[...]
    ps = jnp.sum(h, axis=0, keepdims=True)

    @pl.when(i == 0)
    def _():
        sum_ref[...] = ps

    @pl.when(i != 0)
    def _():
        sum_ref[...] = sum_ref[...] + ps


def _pre_pool(x, w1, b1, w2, b2, w3, b3, w4, b4):
    g = N_NODES // BN
    oneh = pl.BlockSpec((1, DH), lambda i: (0, 0))
    return pl.pallas_call(
        _pre_pool_body,
        grid=(g,),
        in_specs=[
            pl.BlockSpec((BN, D_FEAT), lambda i: (i, 0)),
            pl.BlockSpec((D_FEAT, DH), lambda i: (0, 0)), oneh,
            pl.BlockSpec((DH, DH), lambda i: (0, 0)), oneh,
            pl.BlockSpec((DH, DH), lambda i: (0, 0)), oneh,
            pl.BlockSpec((DH, DH), lambda i: (0, 0)), oneh,
        ],
        out_specs=[oneh],
        out_shape=[jax.ShapeDtypeStruct((1, DH), f32)],
    )(x, w1, b1, w2, b2, w3, b3, w4, b4)[0]


def _final_body(hsum_ref, v0_ref, w1_ref, b1_ref, w2_ref, b2_ref, out_ref):
    pooled = hsum_ref[...] * (1.0 / N_NODES)
    c = _swish(jnp.dot(pooled, w1_ref[...], preferred_element_type=f32) + b1_ref[...])
    coeff = jnp.dot(c, w2_ref[...], preferred_element_type=f32) + b2_ref[...]
    out_ref[...] = v0_ref[...] * coeff


def _final(hsum, v0, w1, b1, w2, b2):
    return pl.pallas_call(
        _final_body,
        out_shape=jax.ShapeDtypeStruct((1, DH), f32),
    )(hsum, v0, w1, b1, w2, b2)


@functools.cache
def _mesh():
    return plsc.VectorSubcoreMesh(core_axis_name="c", subcore_axis_name="s",
                                  num_cores=NC, num_subcores=NS)


def _sc_gather_body(a_hbm, b_hbm, dst_hbm, src_hbm, out_hbm,
                    idxd, idxs, rows_a, rows_b, sem):
    cid = lax.axis_index("c")
    sid = lax.axis_index("s")
    wid = sid * NC + cid
    base = wid * EPW

    def do_chunk(off, k):
        id_d = idxd if k == CHUNK else idxd.at[pl.ds(0, k)]
        id_s = idxs if k == CHUNK else idxs.at[pl.ds(0, k)]
        pltpu.sync_copy(dst_hbm.at[pl.ds(off, k)], id_d)
        pltpu.sync_copy(src_hbm.at[pl.ds(off, k)], id_s)
        cp_a = pltpu.async_copy(a_hbm.at[id_d], rows_a.at[pl.ds(0, k)], sem)
        cp_b = pltpu.async_copy(b_hbm.at[id_s], rows_b.at[pl.ds(0, k)], sem)
        cp_a.wait()
        cp_b.wait()

        def add_row(r, carry):
            for j in range(DP // 16):
                sl = (r, pl.ds(j * 16, 16))
                rows_a[sl] = rows_a[sl] + rows_b[sl]
            return carry

        lax.fori_loop(0, k, add_row, 0)
        pltpu.sync_copy(rows_a.at[pl.ds(0, k)], out_hbm.at[pl.ds(off, k)])

    def loop_body(i, carry):
        do_chunk(base + i * CHUNK, CHUNK)
        return carry

    lax.fori_loop(0, NFULL, loop_body, 0)
    if TAIL:
        do_chunk(base + NFULL * CHUNK, TAIL)


def _gather_edges(a, b, dst, src):
    return pl.kernel(
        _sc_gather_body,
        out_type=jax.ShapeDtypeStruct((N_EDGES, DP), f32),
        mesh=_mesh(),
        scratch_types=[
            pltpu.VMEM((CHUNK,), jnp.int32),
            pltpu.VMEM((CHUNK,), jnp.int32),
            pltpu.VMEM((CHUNK, DP), f32),
            pltpu.VMEM((CHUNK, DP), f32),
            pltpu.SemaphoreType.DMA,
        ],
    )(a, b, dst, src)


def _sc_scatter_body(msg_hbm, dst_hbm, out_hbm, idx, idx_t, rows, zbuf, shared, sem):
    cid = lax.axis_index("c")
    sid = lax.axis_index("s")
    base = sid * EPT
    nbase = cid * OWN

    zvec = jnp.zeros((16,), f32)

    def zrow(r, carry):
        for j in range(DP // 16):
            zbuf[r, pl.ds(j * 16, 16)] = zvec
        return carry

    lax.fori_loop(0, TSTRIPE, zrow, 0)
    pltpu.sync_copy(zbuf, shared.at[pl.ds(sid * TSTRIPE, TSTRIPE)])
    plsc.subcore_barrier()

    def do_chunk(off, k, id_buf):
        pltpu.sync_copy(dst_hbm.at[pl.ds(off, k)], id_buf)
        pltpu.sync_copy(msg_hbm.at[pl.ds(off, k)], rows.at[pl.ds(0, k)])
        for j in range(k // 16):
            sl = pl.ds(j * 16, 16)
            local = id_buf[sl] - nbase
            ok = (local >= 0) & (local < OWN)
            id_buf[sl] = jnp.where(ok, local, TRASH)
        pltpu.sync_copy(rows.at[pl.ds(0, k)], shared.at[id_buf], add=True)

    def loop_body(i, carry):
        do_chunk(base + i * CHUNK, CHUNK, idx)
        return carry

    lax.fori_loop(0, NF2, loop_body, 0)
    if TAIL2:
        do_chunk(base + NF2 * CHUNK, TAIL2, idx_t)

    plsc.subcore_barrier()
    pltpu.sync_copy(shared.at[pl.ds(sid * TSTRIPE, TSTRIPE)],
                    out_hbm.at[cid, pl.ds(sid * TSTRIPE, TSTRIPE)])


def _scatter_msgs(msg, dst):
    return pl.kernel(
        _sc_scatter_body,
        out_type=jax.ShapeDtypeStruct((NC, TROWS, DP), f32),
        mesh=_mesh(),
        scratch_types=[
            pltpu.VMEM((CHUNK,), jnp.int32),
            pltpu.VMEM((TAIL2,), jnp.int32),
            pltpu.VMEM((CHUNK, DP), f32),
            pltpu.VMEM((TSTRIPE, DP), f32),
            pltpu.VMEM_SHARED((TROWS, DP), f32),
            pltpu.SemaphoreType.DMA,
        ],
    )(msg, dst)


def _msg_weights(mp):
    w1 = mp["l1"]["w"]
    wa = _pad2(w1[:, :D_FEAT].T, D_FEAT, DP)
    wb = _pad2(w1[:, D_FEAT:].T, DH, DP)
    b1 = _pad1(mp["l1"]["b"], DP)
    out = {"wa": wa, "wb": wb, "b1": b1}
    for k in ("2", "3"):
        out["w" + k] = _pad2(mp["l" + k]["w"].T, DP, DP)
        out["b" + k] = _pad1(mp["l" + k]["b"], DP)
        out["g" + k] = _pad1(mp["bn" + k]["gamma"], DP)
        out["be" + k] = _pad1(mp["bn" + k]["beta"], DP)
    out["g1"] = _pad1(mp["bn1"]["gamma"], DP)
    out["be1"] = _pad1(mp["bn1"]["beta"], DP)
    out["w4"] = _pad2(mp["l4"]["w"].T, DP, DP)
    out["b4"] = _pad1(mp["l4"]["b"], DP)
    return out


def _gnn_layer(x, v, dst, src, lp):
    mw = _msg_weights(lp["msg"])
    a, b = _node_pre(x, mw["wa"], mw["wb"], mw["b1"])
    e1 = _gather_edges(a, b, dst, src)
    st1 = _s1_stats(e1)
    s2, st2s, st2q = _stage(e1, st1, mw["g1"], mw["be1"], mw["w2"], mw["b2"],
                            pre_swish=True, track_stats=True)
    s3, st3s, st3q = _stage(s2, (st2s, st2q), mw["g2"], mw["be2"], mw["w3"], mw["b3"],
                            pre_swish=False, track_stats=True)
    (msg,) = _stage(s3, (st3s, st3q), mw["g3"], mw["be3"], mw["w4"], mw["b4"],
                    pre_swish=False, track_stats=False, ones_col=True)
    aggs = _scatter_msgs(msg, dst)

    up = lp["upd"]
    w1u = up["l1"]["w"]
    wv = w1u[:, :DH].T
    wx = w1u[:, DH:DH + D_FEAT].T
    wa = _pad2(w1u[:, DH + D_FEAT:].T, DP, DH)
    return _update(
        x, v, aggs,
        wv, wx, wa, _pad1(up["l1"]["b"], DH),
        up["l2"]["w"].T, _pad1(up["l2"]["b"], DH),
        up["l3"]["w"].T, _pad1(up["l3"]["b"], DH),
        up["l4"]["w"].T, _pad1(up["l4"]["b"], D_FEAT),
    )


def kernel(node_feature, edge_index, vectors, params):
    x0 = node_feature[0]
    src = edge_index[0, 0]
    dst = edge_index[0, 1]
    v = x0[:, :DH]
    stacked = jax.tree.map(lambda *a: jnp.stack(a), *params["layers"])

    def _layer_step(xc, lp):
        return _gnn_layer(xc, v, dst, src, lp), None

    x, _ = lax.scan(_layer_step, x0, stacked)

    pp = params["pre"]
    hsum = _pre_pool(
        x,
        pp["l1"]["w"].T, _pad1(pp["l1"]["b"], DH),
        pp["l2"]["w"].T, _pad1(pp["l2"]["b"], DH),
        pp["l3"]["w"].T, _pad1(pp["l3"]["b"], DH),
        pp["l4"]["w"].T, _pad1(pp["l4"]["b"], DH),
    )
    qp = params["post"]
    out = _final(hsum, x0[0:1, :DH],
                 qp["l1"]["w"].T, _pad1(qp["l1"]["b"], DH),
                 qp["l2"]["w"].T, _pad1(qp["l2"]["b"], DH))
    return out.reshape((DH,))
